# baseline (speedup 1.0000x reference)
import jax
import jax.numpy as jnp
from jax.experimental import pallas as pl
from jax.experimental.pallas import tpu as pltpu


def _se_block(x_ref, w1_ref, w2_ref, o_ref):
    x = x_ref[...]
    y = jnp.sum(x, axis=-1, dtype=jnp.float32) * (1.0 / x.shape[-1])
    h = jax.lax.dot_general(y, w1_ref[...], (((1,), (1,)), ((), ())),
                            preferred_element_type=jnp.float32)
    h = jnp.maximum(h, 0.0)
    g = jax.lax.dot_general(h, w2_ref[...], (((1,), (1,)), ((), ())),
                            preferred_element_type=jnp.float32)
    g = jax.nn.sigmoid(g)
    o_ref[...] = x * g[:, :, None]


def _pick_bt(B, C, L, itemsize):
    budget = 8 * 1024 * 1024
    per_batch = C * L * itemsize
    bt = max(1, min(B, budget // per_batch, B // 16 if B >= 16 else B))
    while bt > 1 and B % bt:
        bt -= 1
    return bt


def kernel(x, w1, w2):
    B, C, L = x.shape
    H = w1.shape[0]
    w1 = w1.astype(jnp.float32)
    w2 = w2.astype(jnp.float32)

    bt = _pick_bt(B, C, L, jnp.dtype(x.dtype).itemsize)
    grid = (B // bt,) if B % bt == 0 else (pl.cdiv(B, bt),)

    return pl.pallas_call(
        _se_block,
        out_shape=jax.ShapeDtypeStruct((B, C, L), x.dtype),
        grid=grid,
        in_specs=[
            pl.BlockSpec((bt, C, L), lambda b: (b, 0, 0)),
            pl.BlockSpec((H, C), lambda b: (0, 0)),
            pl.BlockSpec((C, H), lambda b: (0, 0)),
        ],
        out_specs=pl.BlockSpec((bt, C, L), lambda b: (b, 0, 0)),
        compiler_params=pltpu.CompilerParams(
            dimension_semantics=("parallel",),
            vmem_limit_bytes=48 * 1024 * 1024,
        ),
        cost_estimate=pl.CostEstimate(
            flops=2 * B * C * L + 4 * B * C * H,
            transcendentals=B * C,
            bytes_accessed=2 * B * C * L * jnp.dtype(x.dtype).itemsize,
        ),
    )(x, w1, w2)


# speedup vs baseline: 1.0269x; 1.0269x over previous
import jax
import jax.numpy as jnp
from jax.experimental import pallas as pl
from jax.experimental.pallas import tpu as pltpu


def _se_block(x_ref, w1_ref, w2_ref, o_ref):
    x = x_ref[...]
    y = jnp.sum(x, axis=-1, dtype=jnp.float32) * (1.0 / x.shape[-1])
    h = jax.lax.dot_general(y, w1_ref[...], (((1,), (1,)), ((), ())),
                            preferred_element_type=jnp.float32)
    h = jnp.maximum(h, 0.0)
    g = jax.lax.dot_general(h, w2_ref[...], (((1,), (1,)), ((), ())),
                            preferred_element_type=jnp.float32)
    g = jax.nn.sigmoid(g)
    o_ref[...] = x * g[:, :, None]


def _pick_bt(B, C, L, itemsize):
    budget = 16 * 1024 * 1024
    per_batch = C * L * itemsize
    bt = max(1, min(B, budget // per_batch, B // 16 if B >= 16 else B))
    while bt > 1 and B % bt:
        bt -= 1
    return bt


def kernel(x, w1, w2):
    B, C, L = x.shape
    H = w1.shape[0]
    w1 = w1.astype(jnp.float32)
    w2 = w2.astype(jnp.float32)

    bt = _pick_bt(B, C, L, jnp.dtype(x.dtype).itemsize)
    grid = (B // bt,) if B % bt == 0 else (pl.cdiv(B, bt),)

    return pl.pallas_call(
        _se_block,
        out_shape=jax.ShapeDtypeStruct((B, C, L), x.dtype),
        grid=grid,
        in_specs=[
            pl.BlockSpec((bt, C, L), lambda b: (b, 0, 0)),
            pl.BlockSpec((H, C), lambda b: (0, 0)),
            pl.BlockSpec((C, H), lambda b: (0, 0)),
        ],
        out_specs=pl.BlockSpec((bt, C, L), lambda b: (b, 0, 0),
                               pipeline_mode=pl.Buffered(buffer_count=1)),
        compiler_params=pltpu.CompilerParams(
            dimension_semantics=("parallel",),
            vmem_limit_bytes=56 * 1024 * 1024,
        ),
        cost_estimate=pl.CostEstimate(
            flops=2 * B * C * L + 4 * B * C * H,
            transcendentals=B * C,
            bytes_accessed=2 * B * C * L * jnp.dtype(x.dtype).itemsize,
        ),
    )(x, w1, w2)
